# Initial kernel scaffold; baseline (speedup 1.0000x reference)
import jax
import jax.numpy as jnp
from jax.experimental import pallas as pl


def kernel(x, pos_embed):
    raise NotImplementedError("write your pallas kernel here")


# speedup vs baseline: 1.5008x; 1.5008x over previous
import math

import jax
import jax.numpy as jnp
from jax.experimental import pallas as pl


_BS = 512


def _pe_add_kernel(x_ref, pe_ref, o_ref, *, scale):
    o_ref[...] = x_ref[...] + pe_ref[...] * scale


def kernel(x, pos_embed):
    batch, seq_len, d_model = x.shape
    scale = math.sqrt(d_model)
    pe = pos_embed[:seq_len]

    bs = min(_BS, seq_len)
    grid = (seq_len // bs, batch)

    return pl.pallas_call(
        lambda xr, pr, orf: _pe_add_kernel(xr, pr, orf, scale=scale),
        grid=grid,
        in_specs=[
            pl.BlockSpec((1, bs, d_model), lambda s, b: (b, s, 0)),
            pl.BlockSpec((bs, d_model), lambda s, b: (s, 0)),
        ],
        out_specs=pl.BlockSpec((1, bs, d_model), lambda s, b: (b, s, 0)),
        out_shape=jax.ShapeDtypeStruct(x.shape, x.dtype),
    )(x, pe)
